# baseline (speedup 1.0000x reference)
import functools

import jax
import jax.numpy as jnp
from jax import lax
from jax.experimental import pallas as pl
from jax.experimental.pallas import tpu as pltpu
from jax.experimental.pallas import tpu_sc as plsc

NC = 2
NS = 16
L = 16
CH = 128


def _mm1_body(x_ref, w_ref, o_ref):
    o_ref[...] = jnp.dot(x_ref[...], w_ref[...],
                         preferred_element_type=jnp.float32)


def _mid_body(p_ref, b_ref, w_ref, o_ref):
    h = jnp.maximum(p_ref[0] + p_ref[1] + b_ref[...], 0.0)
    o_ref[...] = jnp.dot(h, w_ref[...], preferred_element_type=jnp.float32)


def _head_body(p_ref, b2_ref, wd_ref, bd_ref, mask_ref, wdec_ref, bdec_ref,
               pe_ref, po_ref, o_ref):
    z = jnp.maximum(p_ref[0] + p_ref[1] + b2_ref[...], 0.0)
    f = jnp.maximum(jnp.dot(z, wd_ref[...], preferred_element_type=jnp.float32)
                    + bd_ref[...], 0.0)
    f = jnp.dot(f, mask_ref[...], preferred_element_type=jnp.float32)
    dl = jnp.dot(f, wdec_ref[...], preferred_element_type=jnp.float32) + bdec_ref[...]
    dec = jax.nn.sigmoid(dl)
    fe = f[:, 8:16]
    de = dec[:, 8:16]
    pe = jnp.maximum(pe_ref[...], 0.0)
    po = jnp.maximum(po_ref[...], 0.0)
    o_ref[...] = (jnp.dot(fe * de, pe, preferred_element_type=jnp.float32)
                  + jnp.dot(fe * (1.0 - de), po,
                            preferred_element_type=jnp.float32))


K = 8


def _make_spmm(n_nodes, a_chunks, b_chunks):
    assert a_chunks % K == 0 and b_chunks % K == 0
    cmax = max(a_chunks, b_chunks)
    rows_per_s = n_nodes // NS
    mesh = plsc.VectorSubcoreMesh(core_axis_name="c", subcore_axis_name="s")

    @functools.partial(
        pl.kernel,
        mesh=mesh,
        compiler_params=pltpu.CompilerParams(use_tc_tiling_on_sc=False),
        out_type=jax.ShapeDtypeStruct((NC, n_nodes, L), jnp.float32),
        scratch_types=[
            pltpu.VMEM((cmax, CH), jnp.int32),
            pltpu.VMEM((cmax, CH), jnp.int32),
            pltpu.VMEM((cmax * CH,), jnp.float32),
            pltpu.VMEM((K, CH, L), jnp.float32),
            pltpu.VMEM_SHARED((n_nodes, L), jnp.float32),
        ] + [pltpu.SemaphoreType.DMA] * (2 * K),
    )
    def spmm(table_hbm, src_hbm, dst_hbm, ew_hbm, zeros_hbm, out_hbm,
             src_v, dst_v, ew_v, rows_v, acc_sh, *sems):
        gsem, ssem = sems[:K], sems[K:]
        cid = lax.axis_index("c")
        sid = lax.axis_index("s")
        start = jnp.where(cid == 0, sid * a_chunks,
                          NS * a_chunks + sid * b_chunks)
        n_groups = jnp.where(cid == 0, a_chunks // K, b_chunks // K)

        pltpu.sync_copy(src_hbm.at[pl.ds(start, cmax)], src_v)
        pltpu.sync_copy(dst_hbm.at[pl.ds(start, cmax)], dst_v)
        pltpu.sync_copy(ew_hbm.at[pl.ds(start * CH, cmax * CH)], ew_v)
        pltpu.sync_copy(zeros_hbm.at[pl.ds(sid * rows_per_s, rows_per_s)],
                        acc_sh.at[pl.ds(sid * rows_per_s, rows_per_s)])
        plsc.subcore_barrier()

        def group(g, carry):
            gds = []
            for b in range(K):
                j = g * K + b
                gds.append(pltpu.async_copy(
                    table_hbm.at[src_v.at[j]], rows_v.at[b], gsem[b]))
            sds = []
            for b in range(K):
                j = g * K + b
                gds[b].wait()

                def scale16(t, carry2, b=b, j=j):
                    wv = ew_v[pl.ds(j * CH + t * L, L)]
                    for e in range(L):
                        w = wv.at[jnp.full((L,), e, jnp.int32)].get(
                            mode="promise_in_bounds")
                        r = t * L + e
                        rows_v[b, r] = rows_v[b, r] * w
                    return carry2

                lax.fori_loop(0, CH // L, scale16, 0)
                sds.append(pltpu.async_copy(
                    rows_v.at[b], acc_sh.at[dst_v.at[j]], ssem[b], add=True))
            for b in range(K):
                sds[b].wait()
            return carry

        lax.fori_loop(0, n_groups, group, 0)
        plsc.subcore_barrier()
        pltpu.sync_copy(acc_sh.at[pl.ds(sid * rows_per_s, rows_per_s)],
                        out_hbm.at[cid].at[pl.ds(sid * rows_per_s, rows_per_s)])

    return spmm


def kernel(x, edge_index, edge_weight, W1, b1, W2, b2, Wd, bd, mask, Wdec,
           bdec, pi):
    n, d = x.shape
    hid = W1.shape[1]
    lat = W2.shape[1]
    e = edge_weight.shape[0]
    assert hid == L
    n_pad = -(-n // (NS * 8)) * NS * 8

    pair = -(-e // (NS * CH * 2 * K)) * 2 * K
    a_chunks = max(K, round(pair * 0.65 / K) * K)
    b_chunks = pair - a_chunks
    cmax = max(a_chunks, b_chunks)
    total_rows = NS * pair + cmax
    e_pad = total_rows * CH
    src = jnp.pad(edge_index[0], (0, e_pad - e)).reshape(total_rows, CH)
    dst = jnp.pad(edge_index[1], (0, e_pad - e)).reshape(total_rows, CH)
    ew = jnp.pad(edge_weight, (0, e_pad - e))
    zeros = jnp.zeros((n_pad, L), jnp.float32)
    x_p = jnp.pad(x, ((0, n_pad - n), (0, 0)))

    spmm = _make_spmm(n_pad, a_chunks, b_chunks)

    hw1 = pl.pallas_call(
        _mm1_body, out_shape=jax.ShapeDtypeStruct((n_pad, hid), jnp.float32),
    )(x_p, W1)
    parts1 = spmm(hw1, src, dst, ew, zeros)

    w2p = jnp.pad(W2, ((0, 0), (0, L - lat)))
    hw2 = pl.pallas_call(
        _mid_body, out_shape=jax.ShapeDtypeStruct((n_pad, L), jnp.float32),
    )(parts1, b1.reshape(1, hid), w2p)
    parts2 = spmm(hw2, src, dst, ew, zeros)

    b2p = jnp.pad(b2, (0, L - lat)).reshape(1, L)
    wdp = jnp.pad(Wd, ((0, L - lat), (0, 0)))
    out = pl.pallas_call(
        _head_body,
        out_shape=jax.ShapeDtypeStruct((n_pad, pi.shape[1]), jnp.float32),
    )(parts2, b2p, wdp, bd.reshape(1, -1), mask, Wdec, bdec.reshape(1, -1),
      pi[0::2], pi[1::2])
    return out[:n]


# speedup vs baseline: 1.0525x; 1.0525x over previous
import functools

import jax
import jax.numpy as jnp
from jax import lax
from jax.experimental import pallas as pl
from jax.experimental.pallas import tpu as pltpu
from jax.experimental.pallas import tpu_sc as plsc

NC = 2
NS = 16
L = 16
CH = 256


def _mm1_body(x_ref, w_ref, o_ref):
    o_ref[...] = jnp.dot(x_ref[...], w_ref[...],
                         preferred_element_type=jnp.float32)


def _mid_body(p_ref, b_ref, w_ref, o_ref):
    h = jnp.maximum(p_ref[0] + p_ref[1] + b_ref[...], 0.0)
    o_ref[...] = jnp.dot(h, w_ref[...], preferred_element_type=jnp.float32)


def _head_body(p_ref, b2_ref, wd_ref, bd_ref, mask_ref, wdec_ref, bdec_ref,
               pe_ref, po_ref, o_ref):
    z = jnp.maximum(p_ref[0] + p_ref[1] + b2_ref[...], 0.0)
    f = jnp.maximum(jnp.dot(z, wd_ref[...], preferred_element_type=jnp.float32)
                    + bd_ref[...], 0.0)
    f = jnp.dot(f, mask_ref[...], preferred_element_type=jnp.float32)
    dl = jnp.dot(f, wdec_ref[...], preferred_element_type=jnp.float32) + bdec_ref[...]
    dec = jax.nn.sigmoid(dl)
    fe = f[:, 8:16]
    de = dec[:, 8:16]
    pe = jnp.maximum(pe_ref[...], 0.0)
    po = jnp.maximum(po_ref[...], 0.0)
    o_ref[...] = (jnp.dot(fe * de, pe, preferred_element_type=jnp.float32)
                  + jnp.dot(fe * (1.0 - de), po,
                            preferred_element_type=jnp.float32))


K = 8


def _make_spmm(n_nodes, a_chunks, b_chunks):
    assert a_chunks % K == 0 and b_chunks % K == 0
    cmax = max(a_chunks, b_chunks)
    rows_per_s = n_nodes // NS
    mesh = plsc.VectorSubcoreMesh(core_axis_name="c", subcore_axis_name="s")

    @functools.partial(
        pl.kernel,
        mesh=mesh,
        compiler_params=pltpu.CompilerParams(use_tc_tiling_on_sc=False),
        out_type=jax.ShapeDtypeStruct((NC, n_nodes, L), jnp.float32),
        scratch_types=[
            pltpu.VMEM((cmax, CH), jnp.int32),
            pltpu.VMEM((cmax, CH), jnp.int32),
            pltpu.VMEM((cmax * CH,), jnp.float32),
            pltpu.VMEM((K, CH, L), jnp.float32),
            pltpu.VMEM_SHARED((n_nodes, L), jnp.float32),
        ] + [pltpu.SemaphoreType.DMA] * (2 * K),
    )
    def spmm(table_hbm, src_hbm, dst_hbm, ew_hbm, zeros_hbm, out_hbm,
             src_v, dst_v, ew_v, rows_v, acc_sh, *sems):
        gsem, ssem = sems[:K], sems[K:]
        cid = lax.axis_index("c")
        sid = lax.axis_index("s")
        start = jnp.where(cid == 0, sid * a_chunks,
                          NS * a_chunks + sid * b_chunks)
        n_groups = jnp.where(cid == 0, a_chunks // K, b_chunks // K)

        pltpu.sync_copy(src_hbm.at[pl.ds(start, cmax)], src_v)
        pltpu.sync_copy(dst_hbm.at[pl.ds(start, cmax)], dst_v)
        pltpu.sync_copy(ew_hbm.at[pl.ds(start * CH, cmax * CH)], ew_v)
        pltpu.sync_copy(zeros_hbm.at[pl.ds(sid * rows_per_s, rows_per_s)],
                        acc_sh.at[pl.ds(sid * rows_per_s, rows_per_s)])
        plsc.subcore_barrier()

        def group(g, carry):
            gds = []
            for b in range(K):
                j = g * K + b
                gds.append(pltpu.async_copy(
                    table_hbm.at[src_v.at[j]], rows_v.at[b], gsem[b]))
            sds = []
            for b in range(K):
                j = g * K + b
                gds[b].wait()

                def scale16(t, carry2, b=b, j=j):
                    wv = ew_v[pl.ds(j * CH + t * L, L)]
                    for e in range(L):
                        w = wv.at[jnp.full((L,), e, jnp.int32)].get(
                            mode="promise_in_bounds")
                        r = t * L + e
                        rows_v[b, r] = rows_v[b, r] * w
                    return carry2

                lax.fori_loop(0, CH // L, scale16, 0)
                sds.append(pltpu.async_copy(
                    rows_v.at[b], acc_sh.at[dst_v.at[j]], ssem[b], add=True))
            for b in range(K):
                sds[b].wait()
            return carry

        lax.fori_loop(0, n_groups, group, 0)
        plsc.subcore_barrier()
        pltpu.sync_copy(acc_sh.at[pl.ds(sid * rows_per_s, rows_per_s)],
                        out_hbm.at[cid].at[pl.ds(sid * rows_per_s, rows_per_s)])

    return spmm


def kernel(x, edge_index, edge_weight, W1, b1, W2, b2, Wd, bd, mask, Wdec,
           bdec, pi):
    n, d = x.shape
    hid = W1.shape[1]
    lat = W2.shape[1]
    e = edge_weight.shape[0]
    assert hid == L
    n_pad = -(-n // (NS * 8)) * NS * 8

    pair = -(-e // (NS * CH * 2 * K)) * 2 * K
    a_chunks = max(K, round(pair * 0.7 / K) * K)
    b_chunks = pair - a_chunks
    cmax = max(a_chunks, b_chunks)
    total_rows = NS * pair + cmax
    e_pad = total_rows * CH
    src = jnp.pad(edge_index[0], (0, e_pad - e)).reshape(total_rows, CH)
    dst = jnp.pad(edge_index[1], (0, e_pad - e)).reshape(total_rows, CH)
    ew = jnp.pad(edge_weight, (0, e_pad - e))
    zeros = jnp.zeros((n_pad, L), jnp.float32)
    x_p = jnp.pad(x, ((0, n_pad - n), (0, 0)))

    spmm = _make_spmm(n_pad, a_chunks, b_chunks)

    hw1 = pl.pallas_call(
        _mm1_body, out_shape=jax.ShapeDtypeStruct((n_pad, hid), jnp.float32),
    )(x_p, W1)
    parts1 = spmm(hw1, src, dst, ew, zeros)

    w2p = jnp.pad(W2, ((0, 0), (0, L - lat)))
    hw2 = pl.pallas_call(
        _mid_body, out_shape=jax.ShapeDtypeStruct((n_pad, L), jnp.float32),
    )(parts1, b1.reshape(1, hid), w2p)
    parts2 = spmm(hw2, src, dst, ew, zeros)

    b2p = jnp.pad(b2, (0, L - lat)).reshape(1, L)
    wdp = jnp.pad(Wd, ((0, L - lat), (0, 0)))
    out = pl.pallas_call(
        _head_body,
        out_shape=jax.ShapeDtypeStruct((n_pad, pi.shape[1]), jnp.float32),
    )(parts2, b2p, wdp, bd.reshape(1, -1), mask, Wdec, bdec.reshape(1, -1),
      pi[0::2], pi[1::2])
    return out[:n]
